# baseline (speedup 1.0000x reference)
import functools

import jax
import jax.numpy as jnp
import numpy as np
from jax import lax
from jax.experimental import pallas as pl
from jax.experimental.pallas import tpu as pltpu
from jax.experimental.pallas import tpu_sc as plsc

N = 10000
E = 320000
D = 128

NC = 2
NS = 16
NW = NC * NS
C = 128
_K0 = (E + NW * C - 1) // (NW * C)
K = _K0 + (-_K0 % 4)
K2 = K // 2
NPH = 2
EPAD = NW * K * C
NP = 10240
ZR = NP // NS

EPT = E // NW
PPT = EPAD // NW - EPT
_DPT = (NP - N) // NS
_wids = np.arange(NW, dtype=np.int32)[:, None]
_ks = np.arange(PPT, dtype=np.int32)[None, :]
_PAD_SRC = jnp.asarray((_wids * 317 + _ks) % N)
_PAD_DST = jnp.asarray(N + (_wids % NS) * _DPT + _ks % _DPT)

_mesh = plsc.VectorSubcoreMesh(core_axis_name="c", subcore_axis_name="s")


@functools.partial(
    pl.kernel,
    mesh=_mesh,
    out_type=jax.ShapeDtypeStruct((NC, NP, D), jnp.float32),
    scratch_types=[
        pltpu.VMEM((K2, C), jnp.int32),
        pltpu.VMEM((K2, C), jnp.int32),
        pltpu.VMEM((C, D), jnp.float32),
        pltpu.VMEM((C, D), jnp.float32),
        pltpu.VMEM_SHARED((NP, D), jnp.float32),
        pltpu.SemaphoreType.DMA,
        pltpu.SemaphoreType.DMA,
    ],
)
def _sc_segsum(x_hbm, src_hbm, dst_hbm, zeros_hbm, out_hbm,
               src_v, dst_v, rows_a, rows_b, acc, sem_a, sem_b):
    cid = lax.axis_index("c")
    sid = lax.axis_index("s")
    wid = cid * NS + sid

    pltpu.sync_copy(zeros_hbm, acc.at[pl.ds(sid * ZR, ZR)])
    plsc.subcore_barrier()

    for ph in range(NPH):
        pltpu.sync_copy(src_hbm.at[wid, ph], src_v)
        pltpu.sync_copy(dst_hbm.at[wid, ph], dst_v)

        pltpu.async_copy(x_hbm.at[src_v.at[0]], rows_a, sem_a)

        def body(i, carry):
            j = 2 * i
            pltpu.make_async_copy(x_hbm.at[src_v.at[j]], rows_a, sem_a).wait()
            pltpu.async_copy(x_hbm.at[src_v.at[j + 1]], rows_b, sem_b)
            pltpu.sync_copy(rows_a, acc.at[dst_v.at[j]], add=True)

            pltpu.make_async_copy(x_hbm.at[src_v.at[j + 1]], rows_b,
                                  sem_b).wait()

            @pl.when(j + 2 < K2)
            def _():
                pltpu.async_copy(x_hbm.at[src_v.at[j + 2]], rows_a, sem_a)

            pltpu.sync_copy(rows_b, acc.at[dst_v.at[j + 1]], add=True)
            return carry

        lax.fori_loop(0, K2 // 2, body, 0)

    plsc.subcore_barrier()

    pltpu.sync_copy(acc.at[pl.ds(sid * ZR, ZR)],
                    out_hbm.at[cid, pl.ds(sid * ZR, ZR)])


_R = 2000


def _tc_layer_body(p_ref, x_ref, wr_ref, wo_ref, b_ref, o_ref):
    agg = p_ref[0] + p_ref[1]
    acc = jnp.dot(agg, wr_ref[...], preferred_element_type=jnp.float32)
    acc += jnp.dot(x_ref[...], wo_ref[...], preferred_element_type=jnp.float32)
    o_ref[...] = jnp.maximum(acc + b_ref[...], 0.0)


def _tc_layer(p, x, w_rel, w_root, b):
    return pl.pallas_call(
        _tc_layer_body,
        grid=(N // _R,),
        in_specs=[
            pl.BlockSpec((2, _R, D), lambda i: (0, i, 0)),
            pl.BlockSpec((_R, D), lambda i: (i, 0)),
            pl.BlockSpec((D, D), lambda i: (0, 0)),
            pl.BlockSpec((D, D), lambda i: (0, 0)),
            pl.BlockSpec((1, D), lambda i: (0, 0)),
        ],
        out_specs=pl.BlockSpec((_R, D), lambda i: (i, 0)),
        out_shape=jax.ShapeDtypeStruct((N, D), jnp.float32),
    )(p, x, w_rel, w_root, b)


def _tc_final_body(p_ref, x_ref, wr_ref, wo_ref, b_ref, wl_ref, bl_ref, o_ref):
    agg = p_ref[0] + p_ref[1]
    acc = jnp.dot(agg, wr_ref[...], preferred_element_type=jnp.float32)
    acc += jnp.dot(x_ref[...], wo_ref[...], preferred_element_type=jnp.float32)
    h = jnp.maximum(acc + b_ref[...], 0.0)
    logits = jnp.dot(h, wl_ref[...], preferred_element_type=jnp.float32)
    logits += bl_ref[...]
    m = jnp.max(logits, axis=-1, keepdims=True)
    lse = jnp.log(jnp.sum(jnp.exp(logits - m), axis=-1, keepdims=True)) + m
    o_ref[...] = logits - lse


def _tc_final(p, x, w_rel, w_root, b, w_lin, b_lin):
    return pl.pallas_call(
        _tc_final_body,
        grid=(N // _R,),
        in_specs=[
            pl.BlockSpec((2, _R, D), lambda i: (0, i, 0)),
            pl.BlockSpec((_R, D), lambda i: (i, 0)),
            pl.BlockSpec((D, D), lambda i: (0, 0)),
            pl.BlockSpec((D, D), lambda i: (0, 0)),
            pl.BlockSpec((1, D), lambda i: (0, 0)),
            pl.BlockSpec((D, D), lambda i: (0, 0)),
            pl.BlockSpec((1, D), lambda i: (0, 0)),
        ],
        out_specs=pl.BlockSpec((_R, D), lambda i: (i, 0)),
        out_shape=jax.ShapeDtypeStruct((N, D), jnp.float32),
    )(p, x, w_rel, w_root, b, w_lin, b_lin)


@jax.jit
def kernel(x0, edge_index, W1_rel, b1_rel, W1_root, W2_rel, b2_rel, W2_root,
           W3_rel, b3_rel, W3_root, W_lin, b_lin):
    src_p = jnp.concatenate(
        [edge_index[0].reshape(NW, EPT), _PAD_SRC], axis=1
    ).reshape(NW, NPH, K2, C)
    dst_p = jnp.concatenate(
        [edge_index[1].reshape(NW, EPT), _PAD_DST], axis=1
    ).reshape(NW, NPH, K2, C)
    zeros = jnp.zeros((ZR, D), jnp.float32)

    b1 = b1_rel.reshape(1, D)
    b2 = b2_rel.reshape(1, D)
    b3 = b3_rel.reshape(1, D)
    bl = b_lin.reshape(1, D)

    p1 = _sc_segsum(x0, src_p, dst_p, zeros)
    x1 = _tc_layer(p1, x0, W1_rel, W1_root, b1)
    p2 = _sc_segsum(x1, src_p, dst_p, zeros)
    x2 = _tc_layer(p2, x1, W2_rel, W2_root, b2)
    p3 = _sc_segsum(x2, src_p, dst_p, zeros)
    return _tc_final(p3, x2, W3_rel, W3_root, b3, W_lin, bl)


# speedup vs baseline: 1.1605x; 1.1605x over previous
import functools

import jax
import jax.numpy as jnp
import numpy as np
from jax import lax
from jax.experimental import pallas as pl
from jax.experimental.pallas import tpu as pltpu
from jax.experimental.pallas import tpu_sc as plsc

N = 10000
E = 320000
D = 128

NC = 2
NS = 16
NW = NC * NS
C = 128
_K0 = (E + NW * C - 1) // (NW * C)
K = _K0 + (-_K0 % 4)
K2 = K // 2
NPH = 2
EPAD = NW * K * C
NP = 10240
ZR = NP // NS

EPT = E // NW
PPT = EPAD // NW - EPT
_DPT = (NP - N) // NS
_wids = np.arange(NW, dtype=np.int32)[:, None]
_ks = np.arange(PPT, dtype=np.int32)[None, :]
_PAD_SRC = jnp.asarray((_wids * 317 + _ks) % N)
_PAD_DST = jnp.asarray(N + (_wids % NS) * _DPT + _ks % _DPT)

_mesh = plsc.VectorSubcoreMesh(core_axis_name="c", subcore_axis_name="s")


@functools.partial(
    pl.kernel,
    mesh=_mesh,
    out_type=jax.ShapeDtypeStruct((NC, NP, D), jnp.float32),
    scratch_types=[
        pltpu.VMEM((K2, C), jnp.int32),
        pltpu.VMEM((K2, C), jnp.int32),
        pltpu.VMEM((C, D), jnp.float32),
        pltpu.VMEM((C, D), jnp.float32),
        pltpu.VMEM_SHARED((NP, D), jnp.float32),
        pltpu.SemaphoreType.DMA,
        pltpu.SemaphoreType.DMA,
    ],
)
def _sc_segsum(x_hbm, src_hbm, dst_hbm, zeros_hbm, out_hbm,
               src_v, dst_v, rows_a, rows_b, acc, sem_a, sem_b):
    cid = lax.axis_index("c")
    sid = lax.axis_index("s")
    wid = cid * NS + sid

    pltpu.sync_copy(zeros_hbm, acc.at[pl.ds(sid * ZR, ZR)])
    plsc.subcore_barrier()

    for ph in range(NPH):
        pltpu.sync_copy(src_hbm.at[wid, ph], src_v)
        pltpu.sync_copy(dst_hbm.at[wid, ph], dst_v)

        pltpu.async_copy(x_hbm.at[src_v.at[0]], rows_a, sem_a)
        pltpu.async_copy(x_hbm.at[src_v.at[1]], rows_b, sem_b)

        def body(i, carry):
            j = 2 * i
            pltpu.make_async_copy(x_hbm.at[src_v.at[j]], rows_a, sem_a).wait()
            pltpu.sync_copy(rows_a, acc.at[dst_v.at[j]], add=True)

            @pl.when(j + 2 < K2)
            def _():
                pltpu.async_copy(x_hbm.at[src_v.at[j + 2]], rows_a, sem_a)

            pltpu.make_async_copy(x_hbm.at[src_v.at[j + 1]], rows_b,
                                  sem_b).wait()
            pltpu.sync_copy(rows_b, acc.at[dst_v.at[j + 1]], add=True)

            @pl.when(j + 3 < K2)
            def _():
                pltpu.async_copy(x_hbm.at[src_v.at[j + 3]], rows_b, sem_b)

            return carry

        lax.fori_loop(0, K2 // 2, body, 0)

    plsc.subcore_barrier()

    pltpu.sync_copy(acc.at[pl.ds(sid * ZR, ZR)],
                    out_hbm.at[cid, pl.ds(sid * ZR, ZR)])


_R = 2000


def _tc_layer_body(p_ref, x_ref, wr_ref, wo_ref, b_ref, o_ref):
    agg = p_ref[0] + p_ref[1]
    acc = jnp.dot(agg, wr_ref[...], preferred_element_type=jnp.float32)
    acc += jnp.dot(x_ref[...], wo_ref[...], preferred_element_type=jnp.float32)
    o_ref[...] = jnp.maximum(acc + b_ref[...], 0.0)


def _tc_layer(p, x, w_rel, w_root, b):
    return pl.pallas_call(
        _tc_layer_body,
        grid=(N // _R,),
        in_specs=[
            pl.BlockSpec((2, _R, D), lambda i: (0, i, 0)),
            pl.BlockSpec((_R, D), lambda i: (i, 0)),
            pl.BlockSpec((D, D), lambda i: (0, 0)),
            pl.BlockSpec((D, D), lambda i: (0, 0)),
            pl.BlockSpec((1, D), lambda i: (0, 0)),
        ],
        out_specs=pl.BlockSpec((_R, D), lambda i: (i, 0)),
        out_shape=jax.ShapeDtypeStruct((N, D), jnp.float32),
    )(p, x, w_rel, w_root, b)


def _tc_final_body(p_ref, x_ref, wr_ref, wo_ref, b_ref, wl_ref, bl_ref, o_ref):
    agg = p_ref[0] + p_ref[1]
    acc = jnp.dot(agg, wr_ref[...], preferred_element_type=jnp.float32)
    acc += jnp.dot(x_ref[...], wo_ref[...], preferred_element_type=jnp.float32)
    h = jnp.maximum(acc + b_ref[...], 0.0)
    logits = jnp.dot(h, wl_ref[...], preferred_element_type=jnp.float32)
    logits += bl_ref[...]
    m = jnp.max(logits, axis=-1, keepdims=True)
    lse = jnp.log(jnp.sum(jnp.exp(logits - m), axis=-1, keepdims=True)) + m
    o_ref[...] = logits - lse


def _tc_final(p, x, w_rel, w_root, b, w_lin, b_lin):
    return pl.pallas_call(
        _tc_final_body,
        grid=(N // _R,),
        in_specs=[
            pl.BlockSpec((2, _R, D), lambda i: (0, i, 0)),
            pl.BlockSpec((_R, D), lambda i: (i, 0)),
            pl.BlockSpec((D, D), lambda i: (0, 0)),
            pl.BlockSpec((D, D), lambda i: (0, 0)),
            pl.BlockSpec((1, D), lambda i: (0, 0)),
            pl.BlockSpec((D, D), lambda i: (0, 0)),
            pl.BlockSpec((1, D), lambda i: (0, 0)),
        ],
        out_specs=pl.BlockSpec((_R, D), lambda i: (i, 0)),
        out_shape=jax.ShapeDtypeStruct((N, D), jnp.float32),
    )(p, x, w_rel, w_root, b, w_lin, b_lin)


@jax.jit
def kernel(x0, edge_index, W1_rel, b1_rel, W1_root, W2_rel, b2_rel, W2_root,
           W3_rel, b3_rel, W3_root, W_lin, b_lin):
    src_p = jnp.concatenate(
        [edge_index[0].reshape(NW, EPT), _PAD_SRC], axis=1
    ).reshape(NW, NPH, K2, C)
    dst_p = jnp.concatenate(
        [edge_index[1].reshape(NW, EPT), _PAD_DST], axis=1
    ).reshape(NW, NPH, K2, C)
    zeros = jnp.zeros((ZR, D), jnp.float32)

    b1 = b1_rel.reshape(1, D)
    b2 = b2_rel.reshape(1, D)
    b3 = b3_rel.reshape(1, D)
    bl = b_lin.reshape(1, D)

    p1 = _sc_segsum(x0, src_p, dst_p, zeros)
    x1 = _tc_layer(p1, x0, W1_rel, W1_root, b1)
    p2 = _sc_segsum(x1, src_p, dst_p, zeros)
    x2 = _tc_layer(p2, x1, W2_rel, W2_root, b2)
    p3 = _sc_segsum(x2, src_p, dst_p, zeros)
    return _tc_final(p3, x2, W3_rel, W3_root, b3, W_lin, bl)


# speedup vs baseline: 1.2328x; 1.0623x over previous
import functools

import jax
import jax.numpy as jnp
import numpy as np
from jax import lax
from jax.experimental import pallas as pl
from jax.experimental.pallas import tpu as pltpu
from jax.experimental.pallas import tpu_sc as plsc

N = 10000
E = 320000
D = 128

NC = 2
NS = 16
NW = NC * NS
C = 64
NB = 4
NPH = 4
_K0 = (E + NW * C - 1) // (NW * C)
K = _K0 + (-_K0 % (NB * NPH))
KP = K // NPH
EPAD = NW * K * C
NP = 10240
ZR = NP // NS

EPT = E // NW
PPT = EPAD // NW - EPT
_DPT = (NP - N) // NS
_wids = np.arange(NW, dtype=np.int32)[:, None]
_ks = np.arange(PPT, dtype=np.int32)[None, :]
_PAD_SRC = jnp.asarray((_wids * 317 + _ks) % N)
_PAD_DST = jnp.asarray(N + (_wids % NS) * _DPT + _ks % _DPT)

_mesh = plsc.VectorSubcoreMesh(core_axis_name="c", subcore_axis_name="s")


@functools.partial(
    pl.kernel,
    mesh=_mesh,
    out_type=jax.ShapeDtypeStruct((NC, NP, D), jnp.float32),
    scratch_types=[
        pltpu.VMEM((KP, C), jnp.int32),
        pltpu.VMEM((KP, C), jnp.int32),
        [pltpu.VMEM((C, D), jnp.float32) for _ in range(NB)],
        [pltpu.SemaphoreType.DMA for _ in range(NB)],
        pltpu.VMEM_SHARED((NP, D), jnp.float32),
    ],
)
def _sc_segsum(x_hbm, src_hbm, dst_hbm, zeros_hbm, out_hbm,
               src_v, dst_v, rows, sems, acc):
    cid = lax.axis_index("c")
    sid = lax.axis_index("s")
    wid = cid * NS + sid

    pltpu.sync_copy(zeros_hbm, acc.at[pl.ds(sid * ZR, ZR)])
    plsc.subcore_barrier()

    for ph in range(NPH):
        pltpu.sync_copy(src_hbm.at[wid, ph], src_v)
        pltpu.sync_copy(dst_hbm.at[wid, ph], dst_v)

        for b in range(NB):
            pltpu.async_copy(x_hbm.at[src_v.at[b]], rows[b], sems[b])

        def body(i, carry):
            j = NB * i
            for b in range(NB):
                jj = j + b
                pltpu.make_async_copy(x_hbm.at[src_v.at[jj]], rows[b],
                                      sems[b]).wait()
                pltpu.sync_copy(rows[b], acc.at[dst_v.at[jj]], add=True)

                @pl.when(jj + NB < KP)
                def _():
                    pltpu.async_copy(x_hbm.at[src_v.at[jj + NB]], rows[b],
                                     sems[b])

            return carry

        lax.fori_loop(0, KP // NB, body, 0)

    plsc.subcore_barrier()

    pltpu.sync_copy(acc.at[pl.ds(sid * ZR, ZR)],
                    out_hbm.at[cid, pl.ds(sid * ZR, ZR)])


_R = 2000


def _tc_layer_body(p_ref, x_ref, wr_ref, wo_ref, b_ref, o_ref):
    agg = p_ref[0] + p_ref[1]
    acc = jnp.dot(agg, wr_ref[...], preferred_element_type=jnp.float32)
    acc += jnp.dot(x_ref[...], wo_ref[...], preferred_element_type=jnp.float32)
    o_ref[...] = jnp.maximum(acc + b_ref[...], 0.0)


def _tc_layer(p, x, w_rel, w_root, b):
    return pl.pallas_call(
        _tc_layer_body,
        grid=(N // _R,),
        in_specs=[
            pl.BlockSpec((2, _R, D), lambda i: (0, i, 0)),
            pl.BlockSpec((_R, D), lambda i: (i, 0)),
            pl.BlockSpec((D, D), lambda i: (0, 0)),
            pl.BlockSpec((D, D), lambda i: (0, 0)),
            pl.BlockSpec((1, D), lambda i: (0, 0)),
        ],
        out_specs=pl.BlockSpec((_R, D), lambda i: (i, 0)),
        out_shape=jax.ShapeDtypeStruct((N, D), jnp.float32),
    )(p, x, w_rel, w_root, b)


def _tc_final_body(p_ref, x_ref, wr_ref, wo_ref, b_ref, wl_ref, bl_ref, o_ref):
    agg = p_ref[0] + p_ref[1]
    acc = jnp.dot(agg, wr_ref[...], preferred_element_type=jnp.float32)
    acc += jnp.dot(x_ref[...], wo_ref[...], preferred_element_type=jnp.float32)
    h = jnp.maximum(acc + b_ref[...], 0.0)
    logits = jnp.dot(h, wl_ref[...], preferred_element_type=jnp.float32)
    logits += bl_ref[...]
    m = jnp.max(logits, axis=-1, keepdims=True)
    lse = jnp.log(jnp.sum(jnp.exp(logits - m), axis=-1, keepdims=True)) + m
    o_ref[...] = logits - lse


def _tc_final(p, x, w_rel, w_root, b, w_lin, b_lin):
    return pl.pallas_call(
        _tc_final_body,
        grid=(N // _R,),
        in_specs=[
            pl.BlockSpec((2, _R, D), lambda i: (0, i, 0)),
            pl.BlockSpec((_R, D), lambda i: (i, 0)),
            pl.BlockSpec((D, D), lambda i: (0, 0)),
            pl.BlockSpec((D, D), lambda i: (0, 0)),
            pl.BlockSpec((1, D), lambda i: (0, 0)),
            pl.BlockSpec((D, D), lambda i: (0, 0)),
            pl.BlockSpec((1, D), lambda i: (0, 0)),
        ],
        out_specs=pl.BlockSpec((_R, D), lambda i: (i, 0)),
        out_shape=jax.ShapeDtypeStruct((N, D), jnp.float32),
    )(p, x, w_rel, w_root, b, w_lin, b_lin)


@jax.jit
def kernel(x0, edge_index, W1_rel, b1_rel, W1_root, W2_rel, b2_rel, W2_root,
           W3_rel, b3_rel, W3_root, W_lin, b_lin):
    src_p = jnp.concatenate(
        [edge_index[0].reshape(NW, EPT), _PAD_SRC], axis=1
    ).reshape(NW, NPH, KP, C)
    dst_p = jnp.concatenate(
        [edge_index[1].reshape(NW, EPT), _PAD_DST], axis=1
    ).reshape(NW, NPH, KP, C)
    zeros = jnp.zeros((ZR, D), jnp.float32)

    b1 = b1_rel.reshape(1, D)
    b2 = b2_rel.reshape(1, D)
    b3 = b3_rel.reshape(1, D)
    bl = b_lin.reshape(1, D)

    p1 = _sc_segsum(x0, src_p, dst_p, zeros)
    x1 = _tc_layer(p1, x0, W1_rel, W1_root, b1)
    p2 = _sc_segsum(x1, src_p, dst_p, zeros)
    x2 = _tc_layer(p2, x1, W2_rel, W2_root, b2)
    p3 = _sc_segsum(x2, src_p, dst_p, zeros)
    return _tc_final(p3, x2, W3_rel, W3_root, b3, W_lin, bl)
